# baseline (speedup 1.0000x reference)
import functools

import jax
import jax.numpy as jnp
from jax import lax
from jax.experimental import pallas as pl
from jax.experimental.pallas import tpu as pltpu
from jax.experimental.pallas import tpu_sc as plsc

SEQ = 200
DIM = 32
LANES = 16
NDB = DIM // 8
BT = 128
LC = 4
FC = LC * BT
NCH = SEQ // LC
NBUF = 2


@functools.lru_cache(maxsize=None)
def _build_sc_kernel(n_batch):
    info = plsc.get_sparse_core_info()
    nc, ns = info.num_cores, info.num_subcores
    nw = nc * ns
    n_btiles = n_batch // BT
    bt_per_w = n_btiles // nw
    assert n_batch % BT == 0 and n_btiles % nw == 0

    mesh = plsc.VectorSubcoreMesh(core_axis_name="c", subcore_axis_name="s")

    @functools.partial(
        pl.kernel,
        mesh=mesh,
        compiler_params=pltpu.CompilerParams(
            use_tc_tiling_on_sc=False, needs_layout_passes=False),
        out_type=jax.ShapeDtypeStruct((SEQ, NDB, n_btiles, 8, BT), jnp.float32),
        scratch_types=[
            pltpu.VMEM((BT, SEQ), jnp.int32),
            [pltpu.VMEM((FC,), jnp.int32) for _ in range(NBUF)],
            [pltpu.VMEM((FC, DIM), jnp.float32) for _ in range(NBUF)],
            [pltpu.VMEM((LC, NDB, 8, BT), jnp.float32) for _ in range(NBUF)],
            pltpu.VMEM((SEQ, DIM), jnp.float32),
            [pltpu.SemaphoreType.DMA for _ in range(NBUF)],
            [pltpu.SemaphoreType.DMA for _ in range(NBUF)],
        ],
    )
    def sc_kernel(idx_hbm, term_hbm, pos_hbm, out_hbm,
                  slab_v, idxt_v, rows_v, outs_v, pos_v, gsems, ssems):
        wid = lax.axis_index("s") * nc + lax.axis_index("c")
        pltpu.sync_copy(pos_hbm, pos_v)

        iota = jax.lax.iota(jnp.int32, LANES)
        bvecs = [iota + (bg * LANES) for bg in range(BT // LANES)]

        def transpose_idx(c, b):
            l0 = c * LC
            for li in range(LC):
                lcol = jnp.full((LANES,), l0 + li, dtype=jnp.int32)
                for bg in range(BT // LANES):
                    v = plsc.load_gather(slab_v, [bvecs[bg], lcol])
                    idxt_v[b][pl.ds(li * BT + bg * LANES, LANES)] = v

        def issue_gather(c, b):
            transpose_idx(c, b)
            pltpu.async_copy(term_hbm.at[idxt_v[b]], rows_v[b], gsems[b])

        def wait_gather(b):
            pltpu.make_async_copy(term_hbm.at[idxt_v[b]], rows_v[b], gsems[b]).wait()

        def issue_store(c, bt, b):
            pltpu.async_copy(
                outs_v[b], out_hbm.at[pl.ds(c * LC, LC), :, bt], ssems[b])

        def wait_store(b):
            pltpu.make_async_copy(
                outs_v[b], out_hbm.at[pl.ds(0, LC), :, 0], ssems[b]).wait()

        def compute(c, b):
            l0 = c * LC
            for li in range(LC):
                rvecs = [bv + (li * BT) for bv in bvecs]
                lvec = jnp.full((LANES,), l0 + li, dtype=jnp.int32)

                def d_body(d, _):
                    dvec = jnp.full((LANES,), d, dtype=jnp.int32)
                    p = plsc.load_gather(pos_v, [lvec, dvec])
                    for bg in range(BT // LANES):
                        v = plsc.load_gather(rows_v[b], [rvecs[bg], dvec])
                        outs_v[b][li, d // 8, d % 8,
                                  pl.ds(bg * LANES, LANES)] = v + p
                    return 0

                lax.fori_loop(0, DIM, d_body, 0)

        for t in range(bt_per_w):
            bt = wid * bt_per_w + t
            pltpu.sync_copy(idx_hbm.at[pl.ds(bt * BT, BT)], slab_v)
            issue_gather(0, 0)

            def chunk_body(c, _):
                for b in range(NBUF):
                    c2 = c * NBUF + b
                    bn = (b + 1) % NBUF

                    @pl.when(c2 + 1 < NCH)
                    def _():
                        issue_gather(c2 + 1, bn)

                    wait_gather(b)

                    @pl.when(c2 >= NBUF)
                    def _():
                        wait_store(b)

                    compute(c2, b)
                    issue_store(c2, bt, b)
                return 0

            lax.fori_loop(0, NCH // NBUF, chunk_body, 0)
            for b in range(NBUF):
                wait_store(b)

    return sc_kernel


def kernel(inputs, term_table, pos_table):
    b, l = inputs.shape
    out5 = _build_sc_kernel(b)(inputs, term_table, pos_table)
    return out5.transpose(2, 4, 0, 1, 3).reshape(b, l, DIM)


# speedup vs baseline: 1.6430x; 1.6430x over previous
import functools

import jax
import jax.numpy as jnp
from jax import lax
from jax.experimental import pallas as pl
from jax.experimental.pallas import tpu as pltpu
from jax.experimental.pallas import tpu_sc as plsc

SEQ = 200
DIM = 32
LANES = 16
NDB = DIM // 8
BT = 128
LC = 4
FC = LC * BT
NCH = SEQ // LC
NBUF = 2


@functools.lru_cache(maxsize=None)
def _build_sc_kernel(n_batch):
    info = plsc.get_sparse_core_info()
    nc, ns = info.num_cores, info.num_subcores
    nw = nc * ns
    n_btiles = n_batch // BT
    bt_per_w = n_btiles // nw
    assert n_batch % BT == 0 and n_btiles % nw == 0

    mesh = plsc.VectorSubcoreMesh(core_axis_name="c", subcore_axis_name="s")

    @functools.partial(
        pl.kernel,
        mesh=mesh,
        compiler_params=pltpu.CompilerParams(
            use_tc_tiling_on_sc=False, needs_layout_passes=False),
        out_type=jax.ShapeDtypeStruct((SEQ, NDB, n_btiles, 8, BT), jnp.float32),
        scratch_types=[
            pltpu.VMEM((BT, SEQ), jnp.int32),
            [pltpu.VMEM((FC,), jnp.int32) for _ in range(NBUF)],
            [pltpu.VMEM((FC, DIM), jnp.float32) for _ in range(NBUF)],
            [pltpu.VMEM((LC, NDB, 8, BT), jnp.float32) for _ in range(NBUF)],
            pltpu.VMEM((SEQ, DIM), jnp.float32),
            [pltpu.SemaphoreType.DMA for _ in range(NBUF)],
            [pltpu.SemaphoreType.DMA for _ in range(NBUF)],
        ],
    )
    def sc_kernel(idx_hbm, term_hbm, pos_hbm, out_hbm,
                  slab_v, idxt_v, rows_v, outs_v, pos_v, gsems, ssems):
        wid = lax.axis_index("s") * nc + lax.axis_index("c")
        pltpu.sync_copy(pos_hbm, pos_v)

        iota = jax.lax.iota(jnp.int32, LANES)
        bvecs = [iota + (bg * LANES) for bg in range(BT // LANES)]

        def transpose_idx(c, b):
            l0 = c * LC
            for li in range(LC):
                lcol = jnp.full((LANES,), l0 + li, dtype=jnp.int32)
                for bg in range(BT // LANES):
                    v = plsc.load_gather(slab_v, [bvecs[bg], lcol])
                    idxt_v[b][pl.ds(li * BT + bg * LANES, LANES)] = v

        def issue_gather(c, b):
            transpose_idx(c, b)
            pltpu.async_copy(term_hbm.at[idxt_v[b]], rows_v[b], gsems[b])

        def wait_gather(b):
            pltpu.make_async_copy(term_hbm.at[idxt_v[b]], rows_v[b], gsems[b]).wait()

        def issue_store(c, bt, b):
            pltpu.async_copy(
                outs_v[b], out_hbm.at[pl.ds(c * LC, LC), :, bt], ssems[b])

        def wait_store(b):
            pltpu.make_async_copy(
                outs_v[b], out_hbm.at[pl.ds(0, LC), :, 0], ssems[b]).wait()

        def compute(c, b):
            l0 = c * LC
            for li in range(LC):
                rvecs = [bv + (li * BT) for bv in bvecs]
                lvec = jnp.full((LANES,), l0 + li, dtype=jnp.int32)

                @plsc.parallel_loop(0, DIM, unroll=2)
                def _(d):
                    dvec = jnp.full((LANES,), d, dtype=jnp.int32)
                    p = plsc.load_gather(pos_v, [lvec, dvec])
                    for bg in range(BT // LANES):
                        v = plsc.load_gather(rows_v[b], [rvecs[bg], dvec])
                        outs_v[b][li, d // 8, d % 8,
                                  pl.ds(bg * LANES, LANES)] = v + p

        for t in range(bt_per_w):
            bt = wid * bt_per_w + t
            pltpu.sync_copy(idx_hbm.at[pl.ds(bt * BT, BT)], slab_v)
            issue_gather(0, 0)

            def chunk_body(c, _):
                for b in range(NBUF):
                    c2 = c * NBUF + b
                    bn = (b + 1) % NBUF

                    @pl.when(c2 + 1 < NCH)
                    def _():
                        issue_gather(c2 + 1, bn)

                    wait_gather(b)

                    @pl.when(c2 >= NBUF)
                    def _():
                        wait_store(b)

                    compute(c2, b)
                    issue_store(c2, bt, b)
                return 0

            lax.fori_loop(0, NCH // NBUF, chunk_body, 0)
            for b in range(NBUF):
                wait_store(b)

    return sc_kernel


def kernel(inputs, term_table, pos_table):
    b, l = inputs.shape
    out5 = _build_sc_kernel(b)(inputs, term_table, pos_table)
    return out5.transpose(2, 4, 0, 1, 3).reshape(b, l, DIM)


# speedup vs baseline: 3.8480x; 2.3420x over previous
import functools

import jax
import jax.numpy as jnp
from jax import lax
from jax.experimental import pallas as pl
from jax.experimental.pallas import tpu as pltpu
from jax.experimental.pallas import tpu_sc as plsc

SEQ = 200
SEQP = 201
DIM = 32
LANES = 16
NDB = DIM // 8
BT = 128
BTP = 129
LC = 4
FC = LC * BT
NCH = SEQ // LC
NBUF = 2


@functools.lru_cache(maxsize=None)
def _build_sc_kernel(n_batch):
    info = plsc.get_sparse_core_info()
    nc, ns = info.num_cores, info.num_subcores
    nw = nc * ns
    n_btiles = n_batch // BT
    bt_per_w = n_btiles // nw
    assert n_batch % BT == 0 and n_btiles % nw == 0

    mesh = plsc.VectorSubcoreMesh(core_axis_name="c", subcore_axis_name="s")

    @functools.partial(
        pl.kernel,
        mesh=mesh,
        compiler_params=pltpu.CompilerParams(
            use_tc_tiling_on_sc=False, needs_layout_passes=False),
        out_type=jax.ShapeDtypeStruct((SEQ, NDB, n_btiles, 8, BT), jnp.float32),
        scratch_types=[
            pltpu.VMEM((BT, SEQP), jnp.int32),
            [pltpu.VMEM((FC,), jnp.int32) for _ in range(NBUF)],
            [pltpu.VMEM((FC, DIM), jnp.float32) for _ in range(NBUF)],
            [pltpu.VMEM((LC, NDB, 8, BTP), jnp.float32) for _ in range(NBUF)],
            pltpu.VMEM((SEQ, DIM), jnp.float32),
            [pltpu.SemaphoreType.DMA for _ in range(NBUF)],
            [pltpu.SemaphoreType.DMA for _ in range(NBUF)],
        ],
    )
    def sc_kernel(idx_hbm, term_hbm, pos_hbm, out_hbm,
                  slab_v, idxt_v, rows_v, outs_v, pos_v, gsems, ssems):
        wid = lax.axis_index("s") * nc + lax.axis_index("c")
        pltpu.sync_copy(pos_hbm, pos_v)

        iota = jax.lax.iota(jnp.int32, LANES)
        bvecs = [iota + (bg * LANES) for bg in range(BT // LANES)]
        dblkv = [(iota + h * LANES) // 8 for h in range(2)]
        div = [iota % 8 for h in range(2)]

        def transpose_idx(c, b):
            l0 = c * LC
            for li in range(LC):
                lcol = jnp.full((LANES,), l0 + li, dtype=jnp.int32)
                for bg in range(BT // LANES):
                    v = plsc.load_gather(slab_v, [bvecs[bg], lcol])
                    idxt_v[b][pl.ds(li * BT + bg * LANES, LANES)] = v

        def issue_gather(c, b):
            transpose_idx(c, b)
            pltpu.async_copy(term_hbm.at[idxt_v[b]], rows_v[b], gsems[b])

        def wait_gather(b):
            pltpu.make_async_copy(term_hbm.at[idxt_v[b]], rows_v[b], gsems[b]).wait()

        def issue_store(c, bt, b):
            pltpu.async_copy(
                outs_v[b].at[:, :, :, pl.ds(0, BT)],
                out_hbm.at[pl.ds(c * LC, LC), :, bt], ssems[b])

        def wait_store(b):
            pltpu.make_async_copy(
                outs_v[b].at[:, :, :, pl.ds(0, BT)],
                out_hbm.at[pl.ds(0, LC), :, 0], ssems[b]).wait()

        def compute(c, b):
            l0 = c * LC
            for li in range(LC):
                liv = jnp.full((LANES,), li, dtype=jnp.int32)
                p0 = pos_v[l0 + li, pl.ds(0, LANES)]
                p1 = pos_v[l0 + li, pl.ds(LANES, LANES)]

                @plsc.parallel_loop(0, BT, unroll=4)
                def _(r):
                    row = li * BT + r
                    rv = jnp.full((LANES,), r, dtype=jnp.int32)
                    v0 = rows_v[b][row, pl.ds(0, LANES)] + p0
                    v1 = rows_v[b][row, pl.ds(LANES, LANES)] + p1
                    plsc.store_scatter(outs_v[b], [liv, dblkv[0], div[0], rv], v0)
                    plsc.store_scatter(outs_v[b], [liv, dblkv[1], div[1], rv], v1)

        for t in range(bt_per_w):
            bt = wid * bt_per_w + t
            pltpu.sync_copy(idx_hbm.at[pl.ds(bt * BT, BT)],
                            slab_v.at[:, pl.ds(0, SEQ)])
            issue_gather(0, 0)

            def chunk_body(c, _):
                for b in range(NBUF):
                    c2 = c * NBUF + b
                    bn = (b + 1) % NBUF

                    @pl.when(c2 + 1 < NCH)
                    def _():
                        issue_gather(c2 + 1, bn)

                    wait_gather(b)

                    @pl.when(c2 >= NBUF)
                    def _():
                        wait_store(b)

                    compute(c2, b)
                    issue_store(c2, bt, b)
                return 0

            lax.fori_loop(0, NCH // NBUF, chunk_body, 0)
            for b in range(NBUF):
                wait_store(b)

    return sc_kernel


def kernel(inputs, term_table, pos_table):
    b, l = inputs.shape
    out5 = _build_sc_kernel(b)(inputs, term_table, pos_table)
    return out5.transpose(2, 4, 0, 1, 3).reshape(b, l, DIM)


# speedup vs baseline: 4.1585x; 1.0807x over previous
import functools

import jax
import jax.numpy as jnp
from jax import lax
from jax.experimental import pallas as pl
from jax.experimental.pallas import tpu as pltpu
from jax.experimental.pallas import tpu_sc as plsc

SEQ = 200
SEQP = 201
DIM = 32
LANES = 16
NDB = DIM // 8
BT = 128
BTP = 129
LC = 4
FC = LC * BT
NCH = SEQ // LC
NBUF = 2


TU = 512
TUW = TU * DIM // 128
TVP = TU + 1


@functools.lru_cache(maxsize=None)
def _build_transpose_kernel():
    info = plsc.get_sparse_core_info()
    nc, ns = info.num_cores, info.num_subcores
    nw = nc * ns
    n_rows = 1000000
    n_full = n_rows // TU
    tail = n_rows - n_full * TU
    mesh = plsc.VectorSubcoreMesh(core_axis_name="c", subcore_axis_name="s")

    @functools.partial(
        pl.kernel,
        mesh=mesh,
        compiler_params=pltpu.CompilerParams(
            use_tc_tiling_on_sc=True, needs_layout_passes=False),
        out_type=jax.ShapeDtypeStruct((n_rows * DIM // 128, 128), jnp.float32),
        scratch_types=[
            [pltpu.VMEM((DIM, TVP), jnp.float32) for _ in range(2)],
            [pltpu.VMEM((TUW, 128), jnp.float32) for _ in range(2)],
            pltpu.VMEM((64, 129), jnp.float32),
            [pltpu.SemaphoreType.DMA for _ in range(2)],
            [pltpu.SemaphoreType.DMA for _ in range(2)],
        ],
    )
    def tk(tt_hbm, tail_hbm, out_hbm, tv, wv, tailv, isems, osems):
        wid = lax.axis_index("s") * nc + lax.axis_index("c")
        n_units = jnp.where(wid < n_full % nw, n_full // nw + 1, n_full // nw)

        iota = jax.lax.iota(jnp.int32, LANES)
        dvecs = [(iota + k * LANES) % DIM for k in range(8)]
        rvecs = [(iota + k * LANES) // DIM for k in range(8)]

        def issue_in(k, b):
            u = wid + k * nw
            pltpu.async_copy(tt_hbm.at[:, pl.ds(u * TU, TU)],
                             tv[b].at[:, pl.ds(0, TU)], isems[b])

        def wait_in(b):
            pltpu.make_async_copy(tt_hbm.at[:, pl.ds(0, TU)],
                                  tv[b].at[:, pl.ds(0, TU)], isems[b]).wait()

        def issue_out(k, b):
            u = wid + k * nw
            pltpu.async_copy(wv[b], out_hbm.at[pl.ds(u * TUW, TUW)], osems[b])

        def wait_out(b):
            pltpu.make_async_copy(wv[b], out_hbm.at[pl.ds(0, TUW)],
                                  osems[b]).wait()

        def compute(b, nrows):
            @plsc.parallel_loop(0, nrows, unroll=4)
            def _(i):
                rv = jnp.full((LANES,), 4 * i, dtype=jnp.int32)
                for k in range(8):
                    v = plsc.load_gather(tv[b], [dvecs[k], rvecs[k] + rv])
                    wv[b][i, pl.ds(k * LANES, LANES)] = v

        issue_in(0, 0)

        def unit_body(k, _):
            b = (k % 2).astype(jnp.int32)
            for bb in range(2):
                @pl.when(b == bb)
                def _():
                    @pl.when(k + 1 < n_units)
                    def _():
                        issue_in(k + 1, 1 - bb)
                    wait_in(bb)

                    @pl.when(k >= 2)
                    def _():
                        wait_out(bb)
                    compute(bb, TUW)
                    issue_out(k, bb)
            return 0

        lax.fori_loop(0, n_units, unit_body, 0)
        for bb in range(2):
            @pl.when(n_units > bb)
            def _():
                wait_out(bb)

        @pl.when(wid == 0)
        def _():
            pltpu.sync_copy(tail_hbm, tailv.at[:, pl.ds(0, 128)])
            ntw = tail * DIM // 128

            @plsc.parallel_loop(0, ntw, unroll=4)
            def _(i):
                rv = jnp.full((LANES,), 4 * i, dtype=jnp.int32)
                for k in range(8):
                    v = plsc.load_gather(tailv, [rvecs[k] + rv, dvecs[k]])
                    wv[0][i, pl.ds(k * LANES, LANES)] = v

            pltpu.sync_copy(wv[0].at[pl.ds(0, ntw)],
                            out_hbm.at[pl.ds(n_full * TUW, ntw)])

    return tk


@functools.lru_cache(maxsize=None)
def _build_sc_kernel(n_batch):
    info = plsc.get_sparse_core_info()
    nc, ns = info.num_cores, info.num_subcores
    nw = nc * ns
    n_btiles = n_batch // BT
    bt_per_w = n_btiles // nw
    assert n_batch % BT == 0 and n_btiles % nw == 0

    mesh = plsc.VectorSubcoreMesh(core_axis_name="c", subcore_axis_name="s")

    @functools.partial(
        pl.kernel,
        mesh=mesh,
        compiler_params=pltpu.CompilerParams(
            use_tc_tiling_on_sc=False, needs_layout_passes=False),
        out_type=jax.ShapeDtypeStruct((SEQ, NDB, n_btiles, 8, BT), jnp.float32),
        scratch_types=[
            pltpu.VMEM((BT, SEQP), jnp.int32),
            [pltpu.VMEM((FC,), jnp.int32) for _ in range(NBUF)],
            [pltpu.VMEM((FC, DIM), jnp.float32) for _ in range(NBUF)],
            [pltpu.VMEM((LC, NDB, 8, BTP), jnp.float32) for _ in range(NBUF)],
            pltpu.VMEM((SEQ, DIM), jnp.float32),
            [pltpu.SemaphoreType.DMA for _ in range(NBUF)],
            [pltpu.SemaphoreType.DMA for _ in range(NBUF)],
        ],
    )
    def sc_kernel(idx_hbm, term_hbm, pos_hbm, out_hbm,
                  slab_v, idxt_v, rows_v, outs_v, pos_v, gsems, ssems):
        wid = lax.axis_index("s") * nc + lax.axis_index("c")
        pltpu.sync_copy(pos_hbm, pos_v)

        iota = jax.lax.iota(jnp.int32, LANES)
        bvecs = [iota + (bg * LANES) for bg in range(BT // LANES)]
        dblkv = [(iota + h * LANES) // 8 for h in range(2)]
        div = [iota % 8 for h in range(2)]

        def transpose_idx(c, b):
            l0 = c * LC
            for li in range(LC):
                lcol = jnp.full((LANES,), l0 + li, dtype=jnp.int32)
                for bg in range(BT // LANES):
                    v = plsc.load_gather(slab_v, [bvecs[bg], lcol])
                    idxt_v[b][pl.ds(li * BT + bg * LANES, LANES)] = v

        def issue_gather(c, b):
            transpose_idx(c, b)
            pltpu.async_copy(term_hbm.at[idxt_v[b]], rows_v[b], gsems[b])

        def wait_gather(b):
            pltpu.make_async_copy(term_hbm.at[idxt_v[b]], rows_v[b], gsems[b]).wait()

        def issue_store(c, bt, b):
            pltpu.async_copy(
                outs_v[b].at[:, :, :, pl.ds(0, BT)],
                out_hbm.at[pl.ds(c * LC, LC), :, bt], ssems[b])

        def wait_store(b):
            pltpu.make_async_copy(
                outs_v[b].at[:, :, :, pl.ds(0, BT)],
                out_hbm.at[pl.ds(0, LC), :, 0], ssems[b]).wait()

        def compute(c, b):
            l0 = c * LC
            for li in range(LC):
                liv = jnp.full((LANES,), li, dtype=jnp.int32)
                p0 = pos_v[l0 + li, pl.ds(0, LANES)]
                p1 = pos_v[l0 + li, pl.ds(LANES, LANES)]

                @plsc.parallel_loop(0, BT, unroll=4)
                def _(r):
                    row = li * BT + r
                    rv = jnp.full((LANES,), r, dtype=jnp.int32)
                    v0 = rows_v[b][row, pl.ds(0, LANES)] + p0
                    v1 = rows_v[b][row, pl.ds(LANES, LANES)] + p1
                    plsc.store_scatter(outs_v[b], [liv, dblkv[0], div[0], rv], v0)
                    plsc.store_scatter(outs_v[b], [liv, dblkv[1], div[1], rv], v1)

        for t in range(bt_per_w):
            bt = wid * bt_per_w + t
            pltpu.sync_copy(idx_hbm.at[pl.ds(bt * BT, BT)],
                            slab_v.at[:, pl.ds(0, SEQ)])
            issue_gather(0, 0)

            def chunk_body(c, _):
                for b in range(NBUF):
                    c2 = c * NBUF + b
                    bn = (b + 1) % NBUF

                    @pl.when(c2 + 1 < NCH)
                    def _():
                        issue_gather(c2 + 1, bn)

                    wait_gather(b)

                    @pl.when(c2 >= NBUF)
                    def _():
                        wait_store(b)

                    compute(c2, b)
                    issue_store(c2, bt, b)
                return 0

            lax.fori_loop(0, NCH // NBUF, chunk_body, 0)
            for b in range(NBUF):
                wait_store(b)

    return sc_kernel


def kernel(inputs, term_table, pos_table):
    b, l = inputs.shape
    n_tail = term_table.shape[0] % TU
    tail_pad = jnp.pad(term_table[term_table.shape[0] - n_tail:, :],
                       ((0, 0), (0, 128 - DIM)))
    term_wide = _build_transpose_kernel()(term_table.T, tail_pad)
    term_lin = term_wide.reshape(term_table.shape)
    out5 = _build_sc_kernel(b)(inputs, term_lin, pos_table)
    return out5.transpose(2, 4, 0, 1, 3).reshape(b, l, DIM)


# speedup vs baseline: 4.2315x; 1.0175x over previous
import functools

import jax
import jax.numpy as jnp
from jax import lax
from jax.experimental import pallas as pl
from jax.experimental.pallas import tpu as pltpu
from jax.experimental.pallas import tpu_sc as plsc

SEQ = 200
SEQP = 201
DIM = 32
LANES = 16
NDB = DIM // 8
BT = 128
BTP = 129
LC = 4
FC = LC * BT
NCH = SEQ // LC
NBUF = 2


TU = 512
TUW = TU * DIM // 128
TVP = TU + 1


@functools.lru_cache(maxsize=None)
def _build_transpose_kernel():
    info = plsc.get_sparse_core_info()
    nc, ns = info.num_cores, info.num_subcores
    nw = nc * ns
    n_rows = 1000000
    n_full = n_rows // TU
    tail = n_rows - n_full * TU
    mesh = plsc.VectorSubcoreMesh(core_axis_name="c", subcore_axis_name="s")

    @functools.partial(
        pl.kernel,
        mesh=mesh,
        compiler_params=pltpu.CompilerParams(
            use_tc_tiling_on_sc=True, needs_layout_passes=False),
        out_type=jax.ShapeDtypeStruct((n_rows * DIM // 128, 128), jnp.float32),
        scratch_types=[
            [pltpu.VMEM((DIM, TVP), jnp.float32) for _ in range(3)],
            [pltpu.VMEM((TUW, 128), jnp.float32) for _ in range(3)],
            pltpu.VMEM((64, 129), jnp.float32),
            [pltpu.SemaphoreType.DMA for _ in range(3)],
            [pltpu.SemaphoreType.DMA for _ in range(3)],
        ],
    )
    def tk(tt_hbm, tail_hbm, out_hbm, tv, wv, tailv, isems, osems):
        wid = lax.axis_index("s") * nc + lax.axis_index("c")
        n_units = jnp.where(wid < n_full % nw, n_full // nw + 1, n_full // nw)

        iota = jax.lax.iota(jnp.int32, LANES)
        dvecs = [(iota + k * LANES) % DIM for k in range(8)]
        rvecs = [(iota + k * LANES) // DIM for k in range(8)]

        def issue_in(k, b):
            u = wid + k * nw
            pltpu.async_copy(tt_hbm.at[:, pl.ds(u * TU, TU)],
                             tv[b].at[:, pl.ds(0, TU)], isems[b])

        def wait_in(b):
            pltpu.make_async_copy(tt_hbm.at[:, pl.ds(0, TU)],
                                  tv[b].at[:, pl.ds(0, TU)], isems[b]).wait()

        def issue_out(k, b):
            u = wid + k * nw
            pltpu.async_copy(wv[b], out_hbm.at[pl.ds(u * TUW, TUW)], osems[b])

        def wait_out(b):
            pltpu.make_async_copy(wv[b], out_hbm.at[pl.ds(0, TUW)],
                                  osems[b]).wait()

        def compute(b, nrows):
            @plsc.parallel_loop(0, nrows, unroll=8)
            def _(i):
                rv = jnp.full((LANES,), 4 * i, dtype=jnp.int32)
                for k in range(8):
                    v = plsc.load_gather(tv[b], [dvecs[k], rvecs[k] + rv])
                    wv[b][i, pl.ds(k * LANES, LANES)] = v

        issue_in(0, 0)

        @pl.when(n_units > 1)
        def _():
            issue_in(1, 1)

        def unit_body(k, _):
            b = (k % 3).astype(jnp.int32)
            for bb in range(3):
                @pl.when(b == bb)
                def _():
                    wait_in(bb)

                    @pl.when(k >= 3)
                    def _():
                        wait_out(bb)
                    compute(bb, TUW)
                    issue_out(k, bb)

                    @pl.when(k + 2 < n_units)
                    def _():
                        issue_in(k + 2, (bb + 2) % 3)
            return 0

        lax.fori_loop(0, n_units, unit_body, 0)
        for bb in range(3):
            @pl.when(n_units > bb)
            def _():
                wait_out(bb)

        @pl.when(wid == 0)
        def _():
            pltpu.sync_copy(tail_hbm, tailv.at[:, pl.ds(0, 128)])
            ntw = tail * DIM // 128

            @plsc.parallel_loop(0, ntw, unroll=4)
            def _(i):
                rv = jnp.full((LANES,), 4 * i, dtype=jnp.int32)
                for k in range(8):
                    v = plsc.load_gather(tailv, [rvecs[k] + rv, dvecs[k]])
                    wv[0][i, pl.ds(k * LANES, LANES)] = v

            pltpu.sync_copy(wv[0].at[pl.ds(0, ntw)],
                            out_hbm.at[pl.ds(n_full * TUW, ntw)])

    return tk


@functools.lru_cache(maxsize=None)
def _build_sc_kernel(n_batch):
    info = plsc.get_sparse_core_info()
    nc, ns = info.num_cores, info.num_subcores
    nw = nc * ns
    n_btiles = n_batch // BT
    bt_per_w = n_btiles // nw
    assert n_batch % BT == 0 and n_btiles % nw == 0

    mesh = plsc.VectorSubcoreMesh(core_axis_name="c", subcore_axis_name="s")

    @functools.partial(
        pl.kernel,
        mesh=mesh,
        compiler_params=pltpu.CompilerParams(
            use_tc_tiling_on_sc=False, needs_layout_passes=False),
        out_type=jax.ShapeDtypeStruct((SEQ, NDB, n_btiles, 8, BT), jnp.float32),
        scratch_types=[
            pltpu.VMEM((BT, SEQP), jnp.int32),
            [pltpu.VMEM((FC,), jnp.int32) for _ in range(NBUF)],
            [pltpu.VMEM((FC, DIM), jnp.float32) for _ in range(NBUF)],
            [pltpu.VMEM((LC, NDB, 8, BTP), jnp.float32) for _ in range(NBUF)],
            pltpu.VMEM((SEQ, DIM), jnp.float32),
            [pltpu.SemaphoreType.DMA for _ in range(NBUF)],
            [pltpu.SemaphoreType.DMA for _ in range(NBUF)],
        ],
    )
    def sc_kernel(idx_hbm, term_hbm, pos_hbm, out_hbm,
                  slab_v, idxt_v, rows_v, outs_v, pos_v, gsems, ssems):
        wid = lax.axis_index("s") * nc + lax.axis_index("c")
        pltpu.sync_copy(pos_hbm, pos_v)

        iota = jax.lax.iota(jnp.int32, LANES)
        bvecs = [iota + (bg * LANES) for bg in range(BT // LANES)]
        dblkv = [(iota + h * LANES) // 8 for h in range(2)]
        div = [iota % 8 for h in range(2)]

        def transpose_idx(c, b):
            l0 = c * LC
            for li in range(LC):
                lcol = jnp.full((LANES,), l0 + li, dtype=jnp.int32)
                for bg in range(BT // LANES):
                    v = plsc.load_gather(slab_v, [bvecs[bg], lcol])
                    idxt_v[b][pl.ds(li * BT + bg * LANES, LANES)] = v

        def issue_gather(c, b):
            transpose_idx(c, b)
            pltpu.async_copy(term_hbm.at[idxt_v[b]], rows_v[b], gsems[b])

        def wait_gather(b):
            pltpu.make_async_copy(term_hbm.at[idxt_v[b]], rows_v[b], gsems[b]).wait()

        def issue_store(c, bt, b):
            pltpu.async_copy(
                outs_v[b].at[:, :, :, pl.ds(0, BT)],
                out_hbm.at[pl.ds(c * LC, LC), :, bt], ssems[b])

        def wait_store(b):
            pltpu.make_async_copy(
                outs_v[b].at[:, :, :, pl.ds(0, BT)],
                out_hbm.at[pl.ds(0, LC), :, 0], ssems[b]).wait()

        def compute(c, b):
            l0 = c * LC
            for li in range(LC):
                liv = jnp.full((LANES,), li, dtype=jnp.int32)
                p0 = pos_v[l0 + li, pl.ds(0, LANES)]
                p1 = pos_v[l0 + li, pl.ds(LANES, LANES)]

                @plsc.parallel_loop(0, BT, unroll=4)
                def _(r):
                    row = li * BT + r
                    rv = jnp.full((LANES,), r, dtype=jnp.int32)
                    v0 = rows_v[b][row, pl.ds(0, LANES)] + p0
                    v1 = rows_v[b][row, pl.ds(LANES, LANES)] + p1
                    plsc.store_scatter(outs_v[b], [liv, dblkv[0], div[0], rv], v0)
                    plsc.store_scatter(outs_v[b], [liv, dblkv[1], div[1], rv], v1)

        for t in range(bt_per_w):
            bt = wid * bt_per_w + t
            pltpu.sync_copy(idx_hbm.at[pl.ds(bt * BT, BT)],
                            slab_v.at[:, pl.ds(0, SEQ)])
            issue_gather(0, 0)

            def chunk_body(c, _):
                for b in range(NBUF):
                    c2 = c * NBUF + b
                    bn = (b + 1) % NBUF

                    @pl.when(c2 + 1 < NCH)
                    def _():
                        issue_gather(c2 + 1, bn)

                    wait_gather(b)

                    @pl.when(c2 >= NBUF)
                    def _():
                        wait_store(b)

                    compute(c2, b)
                    issue_store(c2, bt, b)
                return 0

            lax.fori_loop(0, NCH // NBUF, chunk_body, 0)
            for b in range(NBUF):
                wait_store(b)

    return sc_kernel


def kernel(inputs, term_table, pos_table):
    b, l = inputs.shape
    n_tail = term_table.shape[0] % TU
    tail_pad = jnp.pad(term_table[term_table.shape[0] - n_tail:, :],
                       ((0, 0), (0, 128 - DIM)))
    term_wide = _build_transpose_kernel()(term_table.T, tail_pad)
    term_lin = term_wide.reshape(term_table.shape)
    out5 = _build_sc_kernel(b)(inputs, term_lin, pos_table)
    return out5.transpose(2, 4, 0, 1, 3).reshape(b, l, DIM)


# speedup vs baseline: 8.4570x; 1.9986x over previous
import functools

import jax
import jax.numpy as jnp
from jax import lax
from jax.experimental import pallas as pl
from jax.experimental.pallas import tpu as pltpu
from jax.experimental.pallas import tpu_sc as plsc

SEQ = 200
SEQP = 201
DIM = 32
LANES = 16
NDB = DIM // 8
BT = 128
BTP = 129
LC = 4
FC = LC * BT
NCH = SEQ // LC
NBUF = 2


TU = 512
TUW = TU * DIM // 128
TVP = TU + 1


@functools.lru_cache(maxsize=None)
def _build_transpose_kernel():
    info = plsc.get_sparse_core_info()
    nc, ns = info.num_cores, info.num_subcores
    nw = nc * ns
    n_rows = 1000000
    n_full = n_rows // TU
    tail = n_rows - n_full * TU
    mesh = plsc.VectorSubcoreMesh(core_axis_name="c", subcore_axis_name="s")

    @functools.partial(
        pl.kernel,
        mesh=mesh,
        compiler_params=pltpu.CompilerParams(
            use_tc_tiling_on_sc=True, needs_layout_passes=False),
        out_type=jax.ShapeDtypeStruct((n_rows * DIM // 128, 128), jnp.float32),
        scratch_types=[
            [pltpu.VMEM((DIM, TVP), jnp.float32) for _ in range(3)],
            [pltpu.VMEM((TUW, 128), jnp.float32) for _ in range(3)],
            pltpu.VMEM((64, 129), jnp.float32),
            [pltpu.SemaphoreType.DMA for _ in range(3)],
            [pltpu.SemaphoreType.DMA for _ in range(3)],
        ],
    )
    def tk(tt_hbm, tail_hbm, out_hbm, tv, wv, tailv, isems, osems):
        wid = lax.axis_index("s") * nc + lax.axis_index("c")
        n_units = jnp.where(wid < n_full % nw, n_full // nw + 1, n_full // nw)

        iota = jax.lax.iota(jnp.int32, LANES)
        dvecs = [(iota + k * LANES) % DIM for k in range(8)]
        rvecs = [(iota + k * LANES) // DIM for k in range(8)]

        def issue_in(k, b):
            u = wid + k * nw
            for db in range(DIM // 8):
                pltpu.async_copy(
                    tt_hbm.at[pl.ds(db * 8, 8), pl.ds(u * TU, TU)],
                    tv[b].at[pl.ds(db * 8, 8), pl.ds(0, TU)], isems[b])

        def wait_in(b):
            for db in range(DIM // 8):
                pltpu.make_async_copy(
                    tt_hbm.at[pl.ds(0, 8), pl.ds(0, TU)],
                    tv[b].at[pl.ds(0, 8), pl.ds(0, TU)], isems[b]).wait()

        def issue_out(k, b):
            u = wid + k * nw
            pltpu.async_copy(wv[b], out_hbm.at[pl.ds(u * TUW, TUW)], osems[b])

        def wait_out(b):
            pltpu.make_async_copy(wv[b], out_hbm.at[pl.ds(0, TUW)],
                                  osems[b]).wait()

        def compute(b, nrows):
            @plsc.parallel_loop(0, nrows, unroll=8)
            def _(i):
                rv = jnp.full((LANES,), 4 * i, dtype=jnp.int32)
                for k in range(8):
                    v = plsc.load_gather(tv[b], [dvecs[k], rvecs[k] + rv])
                    wv[b][i, pl.ds(k * LANES, LANES)] = v

        issue_in(0, 0)

        @pl.when(n_units > 1)
        def _():
            issue_in(1, 1)

        def unit_body(k, _):
            b = (k % 3).astype(jnp.int32)
            for bb in range(3):
                @pl.when(b == bb)
                def _():
                    wait_in(bb)

                    @pl.when(k >= 3)
                    def _():
                        wait_out(bb)
                    compute(bb, TUW)
                    issue_out(k, bb)

                    @pl.when(k + 2 < n_units)
                    def _():
                        issue_in(k + 2, (bb + 2) % 3)
            return 0

        lax.fori_loop(0, n_units, unit_body, 0)
        for bb in range(3):
            @pl.when(n_units > bb)
            def _():
                wait_out(bb)

        @pl.when(wid == 0)
        def _():
            pltpu.sync_copy(tail_hbm, tailv.at[:, pl.ds(0, 128)])
            ntw = tail * DIM // 128

            @plsc.parallel_loop(0, ntw, unroll=4)
            def _(i):
                rv = jnp.full((LANES,), 4 * i, dtype=jnp.int32)
                for k in range(8):
                    v = plsc.load_gather(tailv, [rvecs[k] + rv, dvecs[k]])
                    wv[0][i, pl.ds(k * LANES, LANES)] = v

            pltpu.sync_copy(wv[0].at[pl.ds(0, ntw)],
                            out_hbm.at[pl.ds(n_full * TUW, ntw)])

    return tk


@functools.lru_cache(maxsize=None)
def _build_sc_kernel(n_batch):
    info = plsc.get_sparse_core_info()
    nc, ns = info.num_cores, info.num_subcores
    nw = nc * ns
    n_btiles = n_batch // BT
    bt_per_w = n_btiles // nw
    assert n_batch % BT == 0 and n_btiles % nw == 0

    mesh = plsc.VectorSubcoreMesh(core_axis_name="c", subcore_axis_name="s")

    @functools.partial(
        pl.kernel,
        mesh=mesh,
        compiler_params=pltpu.CompilerParams(
            use_tc_tiling_on_sc=False, needs_layout_passes=False),
        out_type=jax.ShapeDtypeStruct((SEQ, NDB, n_btiles, 8, BT), jnp.float32),
        scratch_types=[
            pltpu.VMEM((BT, SEQP), jnp.int32),
            [pltpu.VMEM((FC,), jnp.int32) for _ in range(NBUF)],
            [pltpu.VMEM((FC, DIM), jnp.float32) for _ in range(NBUF)],
            [pltpu.VMEM((LC, NDB, 8, BTP), jnp.float32) for _ in range(NBUF)],
            pltpu.VMEM((SEQ, DIM), jnp.float32),
            [pltpu.SemaphoreType.DMA for _ in range(NBUF)],
            [pltpu.SemaphoreType.DMA for _ in range(NBUF)],
        ],
    )
    def sc_kernel(idx_hbm, term_hbm, pos_hbm, out_hbm,
                  slab_v, idxt_v, rows_v, outs_v, pos_v, gsems, ssems):
        wid = lax.axis_index("s") * nc + lax.axis_index("c")
        pltpu.sync_copy(pos_hbm, pos_v)

        iota = jax.lax.iota(jnp.int32, LANES)
        bvecs = [iota + (bg * LANES) for bg in range(BT // LANES)]
        dblkv = [(iota + h * LANES) // 8 for h in range(2)]
        div = [iota % 8 for h in range(2)]

        def transpose_idx(c, b):
            l0 = c * LC
            for li in range(LC):
                lcol = jnp.full((LANES,), l0 + li, dtype=jnp.int32)
                for bg in range(BT // LANES):
                    v = plsc.load_gather(slab_v, [bvecs[bg], lcol])
                    idxt_v[b][pl.ds(li * BT + bg * LANES, LANES)] = v

        def issue_gather(c, b):
            transpose_idx(c, b)
            pltpu.async_copy(term_hbm.at[idxt_v[b]], rows_v[b], gsems[b])

        def wait_gather(b):
            pltpu.make_async_copy(term_hbm.at[idxt_v[b]], rows_v[b], gsems[b]).wait()

        def issue_store(c, bt, b):
            pltpu.async_copy(
                outs_v[b].at[:, :, :, pl.ds(0, BT)],
                out_hbm.at[pl.ds(c * LC, LC), :, bt], ssems[b])

        def wait_store(b):
            pltpu.make_async_copy(
                outs_v[b].at[:, :, :, pl.ds(0, BT)],
                out_hbm.at[pl.ds(0, LC), :, 0], ssems[b]).wait()

        def compute(c, b):
            l0 = c * LC
            for li in range(LC):
                liv = jnp.full((LANES,), li, dtype=jnp.int32)
                p0 = pos_v[l0 + li, pl.ds(0, LANES)]
                p1 = pos_v[l0 + li, pl.ds(LANES, LANES)]

                @plsc.parallel_loop(0, BT, unroll=4)
                def _(r):
                    row = li * BT + r
                    rv = jnp.full((LANES,), r, dtype=jnp.int32)
                    v0 = rows_v[b][row, pl.ds(0, LANES)] + p0
                    v1 = rows_v[b][row, pl.ds(LANES, LANES)] + p1
                    plsc.store_scatter(outs_v[b], [liv, dblkv[0], div[0], rv], v0)
                    plsc.store_scatter(outs_v[b], [liv, dblkv[1], div[1], rv], v1)

        for t in range(bt_per_w):
            bt = wid * bt_per_w + t
            pltpu.sync_copy(idx_hbm.at[pl.ds(bt * BT, BT)],
                            slab_v.at[:, pl.ds(0, SEQ)])
            issue_gather(0, 0)

            def chunk_body(c, _):
                for b in range(NBUF):
                    c2 = c * NBUF + b
                    bn = (b + 1) % NBUF

                    @pl.when(c2 + 1 < NCH)
                    def _():
                        issue_gather(c2 + 1, bn)

                    wait_gather(b)

                    @pl.when(c2 >= NBUF)
                    def _():
                        wait_store(b)

                    compute(c2, b)
                    issue_store(c2, bt, b)
                return 0

            lax.fori_loop(0, NCH // NBUF, chunk_body, 0)
            for b in range(NBUF):
                wait_store(b)

    return sc_kernel


def kernel(inputs, term_table, pos_table):
    b, l = inputs.shape
    n_tail = term_table.shape[0] % TU
    tail_pad = jnp.pad(term_table[term_table.shape[0] - n_tail:, :],
                       ((0, 0), (0, 128 - DIM)))
    term_wide = _build_transpose_kernel()(term_table.T, tail_pad)
    term_lin = term_wide.reshape(term_table.shape)
    out5 = _build_sc_kernel(b)(inputs, term_lin, pos_table)
    return out5.transpose(2, 4, 0, 1, 3).reshape(b, l, DIM)
